# baseline (speedup 1.0000x reference)
import functools

import jax
import jax.numpy as jnp
from jax import lax
from jax.experimental import pallas as pl
from jax.experimental.pallas import tpu as pltpu
from jax.experimental.pallas import tpu_sc as plsc

N_E_ = 8192
E_DIM_ = 64
N_Z_ = 16384

BZ = 256
NWC = 2048
N_CHUNKS = N_E_ // NWC
N_BLOCKS = N_Z_ // BZ
WINDOW_CHUNKS = (N_E_ // 2) // NWC


def _dist_argmin_kernel(z_ref, w_ref, zz_ref, ww_ref, idx_ref, loss_ref):
    zb = z_ref[...]
    zz = zz_ref[...]
    lane = lax.broadcasted_iota(jnp.int32, (BZ, 128), 1)

    def window(first_chunk):
        val = jnp.full((BZ, 128), jnp.inf, jnp.float32)
        pc = jnp.zeros((BZ, 128), jnp.int32)
        for c in range(first_chunk, first_chunk + WINDOW_CHUNKS):
            wc = w_ref[pl.ds(c * NWC, NWC), :]
            ww = ww_ref[:, pl.ds(c * NWC, NWC)]
            mmneg = lax.dot_general(zb, wc, (((1,), (1,)), ((), ())),
                                    preferred_element_type=jnp.float32)
            d = (zz + ww) + mmneg
            for k in range(NWC // 128):
                dk = d[:, k * 128:(k + 1) * 128]
                cond = dk < val
                val = jnp.where(cond, dk, val)
                pc = jnp.where(cond, jnp.int32(c * (NWC // 128) + k), pc)
        m = jnp.min(val, axis=1)
        col = pc * 128 + lane
        i = jnp.min(jnp.where(val == m[:, None], col, N_E_), axis=1)
        return m, i

    m1, i1 = window(0)
    m2, i2 = window(WINDOW_CHUNKS)
    m1q = m1.astype(jnp.bfloat16).astype(jnp.float32)
    upd = m2 < m1q
    best_idx = jnp.where(upd, i2, i1)
    best_exact = jnp.where(upd, m2, m1)
    idx_ref[0, 0, :] = best_idx

    @pl.when(pl.program_id(0) == 0)
    def _zero():
        loss_ref[...] = jnp.zeros_like(loss_ref)

    loss_ref[...] += lax.broadcast_in_dim(jnp.sum(best_exact), (1, 1), ())


def _tc_stage(zm16, W16, zz, ww):
    idx3, loss_sum = pl.pallas_call(
        _dist_argmin_kernel,
        grid=(N_BLOCKS,),
        in_specs=[
            pl.BlockSpec((BZ, E_DIM_), lambda i: (i, 0)),
            pl.BlockSpec((N_E_, E_DIM_), lambda i: (0, 0)),
            pl.BlockSpec((BZ, 1), lambda i: (i, 0)),
            pl.BlockSpec((1, N_E_), lambda i: (0, 0)),
        ],
        out_specs=[
            pl.BlockSpec((1, 1, BZ), lambda i: (i, 0, 0)),
            pl.BlockSpec((1, 1), lambda i: (0, 0)),
        ],
        out_shape=[
            jax.ShapeDtypeStruct((N_BLOCKS, 1, BZ), jnp.int32),
            jax.ShapeDtypeStruct((1, 1), jnp.float32),
        ],
    )(zm16, W16, zz, ww)
    return idx3.reshape(N_Z_), loss_sum[0, 0]


@functools.cache
def _sc_gather_fn():
    info = plsc.get_sparse_core_info()
    nw = info.num_cores * info.num_subcores
    bpw = N_Z_ // nw
    mesh = plsc.VectorSubcoreMesh(core_axis_name="c", subcore_axis_name="s")

    @functools.partial(
        pl.kernel,
        mesh=mesh,
        out_type=jax.ShapeDtypeStruct((N_Z_, E_DIM_), jnp.float32),
        scratch_types=[
            pltpu.VMEM((bpw,), jnp.int32),
            pltpu.VMEM((bpw, E_DIM_), jnp.float32),
            pltpu.SemaphoreType.DMA,
        ],
        compiler_params=pltpu.CompilerParams(use_tc_tiling_on_sc=False),
    )
    def gather(w_hbm, idx_hbm, out_hbm, idx_v, rows_v, sem):
        wid = lax.axis_index("s") * info.num_cores + lax.axis_index("c")
        base = wid * bpw
        pltpu.sync_copy(idx_hbm.at[pl.ds(base, bpw)], idx_v)
        pltpu.async_copy(w_hbm.at[idx_v], rows_v, sem).wait()
        pltpu.sync_copy(rows_v, out_hbm.at[pl.ds(base, bpw)])

    return gather


def kernel(z, W):
    zz = jnp.sum(z ** 2, axis=1)
    ww = jnp.sum(W ** 2, axis=1)
    zm16 = (-2.0 * z).astype(jnp.bfloat16)
    W16 = W.astype(jnp.bfloat16)
    idx, loss_sum = _tc_stage(zm16, W16, zz.reshape(N_Z_, 1), ww.reshape(1, N_E_))
    z_q = _sc_gather_fn()(W, idx)
    loss = 2.0 * loss_sum / (N_Z_ * E_DIM_)
    return loss, z_q, idx


# speedup vs baseline: 1.0867x; 1.0867x over previous
import functools

import jax
import jax.numpy as jnp
from jax import lax
from jax.experimental import pallas as pl
from jax.experimental.pallas import tpu as pltpu
from jax.experimental.pallas import tpu_sc as plsc

N_E_ = 8192
E_DIM_ = 64
N_Z_ = 16384

BZ = 512
NWC = 1024
N_CHUNKS = N_E_ // NWC
N_BLOCKS = N_Z_ // BZ
WINDOW_CHUNKS = (N_E_ // 2) // NWC


def _dist_argmin_kernel(z_ref, w_ref, zz_ref, ww_ref, idx_ref, loss_ref):
    zb = z_ref[...]
    zz = zz_ref[...]
    lane = lax.broadcasted_iota(jnp.int32, (BZ, 128), 1)

    def window(first_chunk):
        val = jnp.full((BZ, 128), jnp.inf, jnp.float32)
        pc = jnp.zeros((BZ, 128), jnp.int32)
        for c in range(first_chunk, first_chunk + WINDOW_CHUNKS):
            wc = w_ref[pl.ds(c * NWC, NWC), :]
            ww = ww_ref[:, pl.ds(c * NWC, NWC)]
            mmneg = lax.dot_general(zb, wc, (((1,), (1,)), ((), ())),
                                    preferred_element_type=jnp.float32)
            d = (zz + ww) + mmneg
            for k in range(NWC // 128):
                dk = d[:, k * 128:(k + 1) * 128]
                cond = dk < val
                val = jnp.where(cond, dk, val)
                pc = jnp.where(cond, jnp.int32(c * (NWC // 128) + k), pc)
        m = jnp.min(val, axis=1)
        col = pc * 128 + lane
        i = jnp.min(jnp.where(val == m[:, None], col, N_E_), axis=1)
        return m, i

    m1, i1 = window(0)
    m2, i2 = window(WINDOW_CHUNKS)
    m1q = m1.astype(jnp.bfloat16).astype(jnp.float32)
    upd = m2 < m1q
    best_idx = jnp.where(upd, i2, i1)
    best_exact = jnp.where(upd, m2, m1)
    idx_ref[0, 0, :] = best_idx

    @pl.when(pl.program_id(0) == 0)
    def _zero():
        loss_ref[...] = jnp.zeros_like(loss_ref)

    loss_ref[...] += lax.broadcast_in_dim(jnp.sum(best_exact), (1, 1), ())


def _tc_stage(zm16, W16, zz, ww):
    idx3, loss_sum = pl.pallas_call(
        _dist_argmin_kernel,
        grid=(N_BLOCKS,),
        in_specs=[
            pl.BlockSpec((BZ, E_DIM_), lambda i: (i, 0)),
            pl.BlockSpec((N_E_, E_DIM_), lambda i: (0, 0)),
            pl.BlockSpec((BZ, 1), lambda i: (i, 0)),
            pl.BlockSpec((1, N_E_), lambda i: (0, 0)),
        ],
        out_specs=[
            pl.BlockSpec((1, 1, BZ), lambda i: (i, 0, 0)),
            pl.BlockSpec((1, 1), lambda i: (0, 0)),
        ],
        out_shape=[
            jax.ShapeDtypeStruct((N_BLOCKS, 1, BZ), jnp.int32),
            jax.ShapeDtypeStruct((1, 1), jnp.float32),
        ],
    )(zm16, W16, zz, ww)
    return idx3.reshape(N_Z_), loss_sum[0, 0]


@functools.cache
def _sc_gather_fn():
    info = plsc.get_sparse_core_info()
    nw = info.num_cores * info.num_subcores
    bpw = N_Z_ // nw
    mesh = plsc.VectorSubcoreMesh(core_axis_name="c", subcore_axis_name="s")

    @functools.partial(
        pl.kernel,
        mesh=mesh,
        out_type=jax.ShapeDtypeStruct((N_Z_, E_DIM_), jnp.float32),
        scratch_types=[
            pltpu.VMEM((bpw,), jnp.int32),
            pltpu.VMEM((bpw, E_DIM_), jnp.float32),
            pltpu.SemaphoreType.DMA,
        ],
        compiler_params=pltpu.CompilerParams(use_tc_tiling_on_sc=False),
    )
    def gather(w_hbm, idx_hbm, out_hbm, idx_v, rows_v, sem):
        wid = lax.axis_index("s") * info.num_cores + lax.axis_index("c")
        base = wid * bpw
        pltpu.sync_copy(idx_hbm.at[pl.ds(base, bpw)], idx_v)
        pltpu.async_copy(w_hbm.at[idx_v], rows_v, sem).wait()
        pltpu.sync_copy(rows_v, out_hbm.at[pl.ds(base, bpw)])

    return gather


def kernel(z, W):
    zz = jnp.sum(z ** 2, axis=1)
    ww = jnp.sum(W ** 2, axis=1)
    zm16 = (-2.0 * z).astype(jnp.bfloat16)
    W16 = W.astype(jnp.bfloat16)
    idx, loss_sum = _tc_stage(zm16, W16, zz.reshape(N_Z_, 1), ww.reshape(1, N_E_))
    z_q = _sc_gather_fn()(W, idx)
    loss = 2.0 * loss_sum / (N_Z_ * E_DIM_)
    return loss, z_q, idx


# speedup vs baseline: 1.1150x; 1.0261x over previous
import functools

import jax
import jax.numpy as jnp
from jax import lax
from jax.experimental import pallas as pl
from jax.experimental.pallas import tpu as pltpu
from jax.experimental.pallas import tpu_sc as plsc

N_E_ = 8192
E_DIM_ = 64
N_Z_ = 16384

BZ = 1024
NWC = 1024
N_CHUNKS = N_E_ // NWC
N_BLOCKS = N_Z_ // BZ
WINDOW_CHUNKS = (N_E_ // 2) // NWC


def _dist_argmin_kernel(z_ref, w_ref, zz_ref, ww_ref, idx_ref, loss_ref):
    zb = z_ref[...]
    zz = zz_ref[...]
    lane = lax.broadcasted_iota(jnp.int32, (BZ, 128), 1)

    def window(first_chunk):
        val = jnp.full((BZ, 128), jnp.inf, jnp.float32)
        pc = jnp.zeros((BZ, 128), jnp.int32)
        for c in range(first_chunk, first_chunk + WINDOW_CHUNKS):
            wc = w_ref[pl.ds(c * NWC, NWC), :]
            ww = ww_ref[:, pl.ds(c * NWC, NWC)]
            mmneg = lax.dot_general(zb, wc, (((1,), (1,)), ((), ())),
                                    preferred_element_type=jnp.float32)
            d = (zz + ww) + mmneg
            for k in range(NWC // 128):
                dk = d[:, k * 128:(k + 1) * 128]
                cond = dk < val
                val = jnp.where(cond, dk, val)
                pc = jnp.where(cond, jnp.int32(c * (NWC // 128) + k), pc)
        m = jnp.min(val, axis=1)
        col = pc * 128 + lane
        i = jnp.min(jnp.where(val == m[:, None], col, N_E_), axis=1)
        return m, i

    m1, i1 = window(0)
    m2, i2 = window(WINDOW_CHUNKS)
    m1q = m1.astype(jnp.bfloat16).astype(jnp.float32)
    upd = m2 < m1q
    best_idx = jnp.where(upd, i2, i1)
    best_exact = jnp.where(upd, m2, m1)
    idx_ref[0, 0, :] = best_idx

    @pl.when(pl.program_id(0) == 0)
    def _zero():
        loss_ref[...] = jnp.zeros_like(loss_ref)

    loss_ref[...] += lax.broadcast_in_dim(jnp.sum(best_exact), (1, 1), ())


def _tc_stage(zm16, W16, zz, ww):
    idx3, loss_sum = pl.pallas_call(
        _dist_argmin_kernel,
        grid=(N_BLOCKS,),
        in_specs=[
            pl.BlockSpec((BZ, E_DIM_), lambda i: (i, 0)),
            pl.BlockSpec((N_E_, E_DIM_), lambda i: (0, 0)),
            pl.BlockSpec((BZ, 1), lambda i: (i, 0)),
            pl.BlockSpec((1, N_E_), lambda i: (0, 0)),
        ],
        out_specs=[
            pl.BlockSpec((1, 1, BZ), lambda i: (i, 0, 0)),
            pl.BlockSpec((1, 1), lambda i: (0, 0)),
        ],
        out_shape=[
            jax.ShapeDtypeStruct((N_BLOCKS, 1, BZ), jnp.int32),
            jax.ShapeDtypeStruct((1, 1), jnp.float32),
        ],
    )(zm16, W16, zz, ww)
    return idx3.reshape(N_Z_), loss_sum[0, 0]


@functools.cache
def _sc_gather_fn():
    info = plsc.get_sparse_core_info()
    nw = info.num_cores * info.num_subcores
    bpw = N_Z_ // nw
    mesh = plsc.VectorSubcoreMesh(core_axis_name="c", subcore_axis_name="s")

    @functools.partial(
        pl.kernel,
        mesh=mesh,
        out_type=jax.ShapeDtypeStruct((N_Z_, E_DIM_), jnp.float32),
        scratch_types=[
            pltpu.VMEM((bpw,), jnp.int32),
            pltpu.VMEM((bpw, E_DIM_), jnp.float32),
            pltpu.SemaphoreType.DMA,
        ],
        compiler_params=pltpu.CompilerParams(use_tc_tiling_on_sc=False),
    )
    def gather(w_hbm, idx_hbm, out_hbm, idx_v, rows_v, sem):
        wid = lax.axis_index("s") * info.num_cores + lax.axis_index("c")
        base = wid * bpw
        pltpu.sync_copy(idx_hbm.at[pl.ds(base, bpw)], idx_v)
        pltpu.async_copy(w_hbm.at[idx_v], rows_v, sem).wait()
        pltpu.sync_copy(rows_v, out_hbm.at[pl.ds(base, bpw)])

    return gather


def kernel(z, W):
    zz = jnp.sum(z ** 2, axis=1)
    ww = jnp.sum(W ** 2, axis=1)
    zm16 = (-2.0 * z).astype(jnp.bfloat16)
    W16 = W.astype(jnp.bfloat16)
    idx, loss_sum = _tc_stage(zm16, W16, zz.reshape(N_Z_, 1), ww.reshape(1, N_E_))
    z_q = _sc_gather_fn()(W, idx)
    loss = 2.0 * loss_sum / (N_Z_ * E_DIM_)
    return loss, z_q, idx


# speedup vs baseline: 1.1395x; 1.0220x over previous
import functools

import jax
import jax.numpy as jnp
from jax import lax
from jax.experimental import pallas as pl
from jax.experimental.pallas import tpu as pltpu
from jax.experimental.pallas import tpu_sc as plsc

N_E_ = 8192
E_DIM_ = 64
N_Z_ = 16384

BZ = 2048
NWC = 1024
N_CHUNKS = N_E_ // NWC
N_BLOCKS = N_Z_ // BZ
WINDOW_CHUNKS = (N_E_ // 2) // NWC


def _dist_argmin_kernel(z_ref, w_ref, zz_ref, ww_ref, idx_ref, loss_ref):
    zb = z_ref[...]
    zz = zz_ref[...]
    lane = lax.broadcasted_iota(jnp.int32, (BZ, 128), 1)

    def window(first_chunk):
        val = jnp.full((BZ, 128), jnp.inf, jnp.float32)
        pc = jnp.zeros((BZ, 128), jnp.int32)
        for c in range(first_chunk, first_chunk + WINDOW_CHUNKS):
            wc = w_ref[pl.ds(c * NWC, NWC), :]
            ww = ww_ref[:, pl.ds(c * NWC, NWC)]
            mmneg = lax.dot_general(zb, wc, (((1,), (1,)), ((), ())),
                                    preferred_element_type=jnp.float32)
            d = (zz + ww) + mmneg
            for k in range(NWC // 128):
                dk = d[:, k * 128:(k + 1) * 128]
                cond = dk < val
                val = jnp.where(cond, dk, val)
                pc = jnp.where(cond, jnp.int32(c * (NWC // 128) + k), pc)
        m = jnp.min(val, axis=1)
        col = pc * 128 + lane
        i = jnp.min(jnp.where(val == m[:, None], col, N_E_), axis=1)
        return m, i

    m1, i1 = window(0)
    m2, i2 = window(WINDOW_CHUNKS)
    m1q = m1.astype(jnp.bfloat16).astype(jnp.float32)
    upd = m2 < m1q
    best_idx = jnp.where(upd, i2, i1)
    best_exact = jnp.where(upd, m2, m1)
    idx_ref[0, 0, :] = best_idx

    @pl.when(pl.program_id(0) == 0)
    def _zero():
        loss_ref[...] = jnp.zeros_like(loss_ref)

    loss_ref[...] += lax.broadcast_in_dim(jnp.sum(best_exact), (1, 1), ())


def _tc_stage(zm16, W16, zz, ww):
    idx3, loss_sum = pl.pallas_call(
        _dist_argmin_kernel,
        grid=(N_BLOCKS,),
        in_specs=[
            pl.BlockSpec((BZ, E_DIM_), lambda i: (i, 0)),
            pl.BlockSpec((N_E_, E_DIM_), lambda i: (0, 0)),
            pl.BlockSpec((BZ, 1), lambda i: (i, 0)),
            pl.BlockSpec((1, N_E_), lambda i: (0, 0)),
        ],
        out_specs=[
            pl.BlockSpec((1, 1, BZ), lambda i: (i, 0, 0)),
            pl.BlockSpec((1, 1), lambda i: (0, 0)),
        ],
        out_shape=[
            jax.ShapeDtypeStruct((N_BLOCKS, 1, BZ), jnp.int32),
            jax.ShapeDtypeStruct((1, 1), jnp.float32),
        ],
    )(zm16, W16, zz, ww)
    return idx3.reshape(N_Z_), loss_sum[0, 0]


@functools.cache
def _sc_gather_fn():
    info = plsc.get_sparse_core_info()
    nw = info.num_cores * info.num_subcores
    bpw = N_Z_ // nw
    mesh = plsc.VectorSubcoreMesh(core_axis_name="c", subcore_axis_name="s")

    @functools.partial(
        pl.kernel,
        mesh=mesh,
        out_type=jax.ShapeDtypeStruct((N_Z_, E_DIM_), jnp.float32),
        scratch_types=[
            pltpu.VMEM((bpw,), jnp.int32),
            pltpu.VMEM((bpw, E_DIM_), jnp.float32),
            pltpu.SemaphoreType.DMA,
        ],
        compiler_params=pltpu.CompilerParams(use_tc_tiling_on_sc=False),
    )
    def gather(w_hbm, idx_hbm, out_hbm, idx_v, rows_v, sem):
        wid = lax.axis_index("s") * info.num_cores + lax.axis_index("c")
        base = wid * bpw
        pltpu.sync_copy(idx_hbm.at[pl.ds(base, bpw)], idx_v)
        pltpu.async_copy(w_hbm.at[idx_v], rows_v, sem).wait()
        pltpu.sync_copy(rows_v, out_hbm.at[pl.ds(base, bpw)])

    return gather


def kernel(z, W):
    zz = jnp.sum(z ** 2, axis=1)
    ww = jnp.sum(W ** 2, axis=1)
    zm16 = (-2.0 * z).astype(jnp.bfloat16)
    W16 = W.astype(jnp.bfloat16)
    idx, loss_sum = _tc_stage(zm16, W16, zz.reshape(N_Z_, 1), ww.reshape(1, N_E_))
    z_q = _sc_gather_fn()(W, idx)
    loss = 2.0 * loss_sum / (N_Z_ * E_DIM_)
    return loss, z_q, idx


# speedup vs baseline: 1.1702x; 1.0269x over previous
import functools

import jax
import jax.numpy as jnp
from jax import lax
from jax.experimental import pallas as pl
from jax.experimental.pallas import tpu as pltpu
from jax.experimental.pallas import tpu_sc as plsc

N_E_ = 8192
E_DIM_ = 64
N_Z_ = 16384

BZ = 4096
NWC = 512
N_CHUNKS = N_E_ // NWC
N_BLOCKS = N_Z_ // BZ
WINDOW_CHUNKS = (N_E_ // 2) // NWC


def _dist_argmin_kernel(z_ref, w_ref, zz_ref, ww_ref, idx_ref, loss_ref):
    zb = z_ref[...]
    zz = zz_ref[...]
    lane = lax.broadcasted_iota(jnp.int32, (BZ, 128), 1)

    def window(first_chunk):
        val = jnp.full((BZ, 128), jnp.inf, jnp.float32)
        pc = jnp.zeros((BZ, 128), jnp.int32)
        for c in range(first_chunk, first_chunk + WINDOW_CHUNKS):
            wc = w_ref[pl.ds(c * NWC, NWC), :]
            ww = ww_ref[:, pl.ds(c * NWC, NWC)]
            mmneg = lax.dot_general(zb, wc, (((1,), (1,)), ((), ())),
                                    preferred_element_type=jnp.float32)
            d = (zz + ww) + mmneg
            for k in range(NWC // 128):
                dk = d[:, k * 128:(k + 1) * 128]
                cond = dk < val
                val = jnp.where(cond, dk, val)
                pc = jnp.where(cond, jnp.int32(c * (NWC // 128) + k), pc)
        m = jnp.min(val, axis=1)
        col = pc * 128 + lane
        i = jnp.min(jnp.where(val == m[:, None], col, N_E_), axis=1)
        return m, i

    m1, i1 = window(0)
    m2, i2 = window(WINDOW_CHUNKS)
    m1q = m1.astype(jnp.bfloat16).astype(jnp.float32)
    upd = m2 < m1q
    best_idx = jnp.where(upd, i2, i1)
    best_exact = jnp.where(upd, m2, m1)
    idx_ref[0, 0, :] = best_idx

    @pl.when(pl.program_id(0) == 0)
    def _zero():
        loss_ref[...] = jnp.zeros_like(loss_ref)

    loss_ref[...] += lax.broadcast_in_dim(jnp.sum(best_exact), (1, 1), ())


def _tc_stage(zm16, W16, zz, ww):
    idx3, loss_sum = pl.pallas_call(
        _dist_argmin_kernel,
        grid=(N_BLOCKS,),
        in_specs=[
            pl.BlockSpec((BZ, E_DIM_), lambda i: (i, 0)),
            pl.BlockSpec((N_E_, E_DIM_), lambda i: (0, 0)),
            pl.BlockSpec((BZ, 1), lambda i: (i, 0)),
            pl.BlockSpec((1, N_E_), lambda i: (0, 0)),
        ],
        out_specs=[
            pl.BlockSpec((1, 1, BZ), lambda i: (i, 0, 0)),
            pl.BlockSpec((1, 1), lambda i: (0, 0)),
        ],
        out_shape=[
            jax.ShapeDtypeStruct((N_BLOCKS, 1, BZ), jnp.int32),
            jax.ShapeDtypeStruct((1, 1), jnp.float32),
        ],
    )(zm16, W16, zz, ww)
    return idx3.reshape(N_Z_), loss_sum[0, 0]


@functools.cache
def _sc_gather_fn():
    info = plsc.get_sparse_core_info()
    nw = info.num_cores * info.num_subcores
    bpw = N_Z_ // nw
    mesh = plsc.VectorSubcoreMesh(core_axis_name="c", subcore_axis_name="s")

    @functools.partial(
        pl.kernel,
        mesh=mesh,
        out_type=jax.ShapeDtypeStruct((N_Z_, E_DIM_), jnp.float32),
        scratch_types=[
            pltpu.VMEM((bpw,), jnp.int32),
            pltpu.VMEM((bpw, E_DIM_), jnp.float32),
            pltpu.SemaphoreType.DMA,
        ],
        compiler_params=pltpu.CompilerParams(use_tc_tiling_on_sc=False),
    )
    def gather(w_hbm, idx_hbm, out_hbm, idx_v, rows_v, sem):
        wid = lax.axis_index("s") * info.num_cores + lax.axis_index("c")
        base = wid * bpw
        pltpu.sync_copy(idx_hbm.at[pl.ds(base, bpw)], idx_v)
        pltpu.async_copy(w_hbm.at[idx_v], rows_v, sem).wait()
        pltpu.sync_copy(rows_v, out_hbm.at[pl.ds(base, bpw)])

    return gather


def kernel(z, W):
    zz = jnp.sum(z ** 2, axis=1)
    ww = jnp.sum(W ** 2, axis=1)
    zm16 = (-2.0 * z).astype(jnp.bfloat16)
    W16 = W.astype(jnp.bfloat16)
    idx, loss_sum = _tc_stage(zm16, W16, zz.reshape(N_Z_, 1), ww.reshape(1, N_E_))
    z_q = _sc_gather_fn()(W, idx)
    loss = 2.0 * loss_sum / (N_Z_ * E_DIM_)
    return loss, z_q, idx
